# baseline (speedup 1.0000x reference)
import functools

import jax
import jax.numpy as jnp
from jax import lax
from jax.experimental import pallas as pl
from jax.experimental.pallas import tpu as pltpu
from jax.experimental.pallas import tpu_sc as plsc

N, C, H, W = 4, 192, 224, 224
HO, WO = 224, 224
P = N * HO * WO
HW = H * W
L = 16
NC, NS = 2, 16
NW = NC * NS
PPT = P // NW
TPB = (HO * WO) // PPT
CH = 64
NCHUNK = PPT // CH
NPAIR = NCHUNK // 2
CCH = C // L
TBLK = 3584


def _lane_bcast(v, j):
    idx = jnp.full((L,), j, jnp.int32)
    return lax.gather(
        v, idx[:, None],
        dimension_numbers=lax.GatherDimensionNumbers(
            offset_dims=(), collapsed_slice_dims=(0,), start_index_map=(0,)),
        slice_sizes=(1,), mode=lax.GatherScatterMode.PROMISE_IN_BOUNDS)


def _eye_bf16():
    ii = lax.broadcasted_iota(jnp.int32, (C, C), 0)
    jj = lax.broadcasted_iota(jnp.int32, (C, C), 1)
    return (ii == jj).astype(jnp.bfloat16)


def _transpose_to_rows(x3):
    def body(x_ref, o_ref):
        xb = x_ref[0].astype(jnp.bfloat16)
        y = lax.dot_general(xb, _eye_bf16(), (((0,), (0,)), ((), ())),
                            preferred_element_type=jnp.float32)
        o_ref[...] = y.astype(jnp.bfloat16)[None]
    return pl.pallas_call(
        body,
        grid=(N, HW // TBLK),
        in_specs=[pl.BlockSpec((1, C, TBLK), lambda n, t: (n, 0, t))],
        out_specs=pl.BlockSpec((1, TBLK, C), lambda n, t: (n, t, 0)),
        out_shape=jax.ShapeDtypeStruct((N, HW, C), jnp.bfloat16),
    )(x3)


def _transpose_to_planes(y3):
    def body(y_ref, o_ref):
        yb = y_ref[0]
        y = lax.dot_general(_eye_bf16(), yb, (((1,), (1,)), ((), ())),
                            preferred_element_type=jnp.float32)
        o_ref[...] = y[None]
    return pl.pallas_call(
        body,
        grid=(N, HW // TBLK),
        in_specs=[pl.BlockSpec((1, TBLK, C), lambda n, t: (n, t, 0))],
        out_specs=pl.BlockSpec((1, C, TBLK), lambda n, t: (n, 0, t)),
        out_shape=jax.ShapeDtypeStruct((N, C, HW), jnp.float32),
    )(y3)


def _make_sc_kernel():
    mesh = plsc.VectorSubcoreMesh(core_axis_name="c", subcore_axis_name="s")

    buf = lambda shape, dt: pltpu.VMEM(shape, dt)
    bufset = lambda: [
        buf((2, CH), jnp.float32),
        buf((CH,), jnp.int32), buf((CH,), jnp.int32),
        buf((CH,), jnp.int32), buf((CH,), jnp.int32),
        buf((CH,), jnp.float32), buf((CH,), jnp.float32),
        buf((CH,), jnp.float32), buf((CH,), jnp.float32),
        buf((CH, C), jnp.bfloat16), buf((CH, C), jnp.bfloat16),
        buf((CH, C), jnp.bfloat16), buf((CH, C), jnp.bfloat16),
        buf((CH, C), jnp.bfloat16),
        pltpu.SemaphoreType.DMA,
        pltpu.SemaphoreType.DMA,
    ]

    @functools.partial(
        pl.kernel,
        mesh=mesh,
        compiler_params=pltpu.CompilerParams(
            use_tc_tiling_on_sc=False, needs_layout_passes=False),
        out_type=jax.ShapeDtypeStruct((P, C), jnp.bfloat16),
        scratch_types=bufset() + bufset(),
    )
    def grid_sample_sc(xt_hbm, g2_hbm, out_hbm, *scr):
        A, B = scr[:16], scr[16:]
        wid = lax.axis_index("s") * NC + lax.axis_index("c")
        nimg = wid // TPB
        pbase = wid * PPT

        def stage_and_fire(ci, S):
            (g_v, i00, i01, i10, i11, w00, w01, w10, w11,
             r00, r01, r10, r11, _out, sem, _osem) = S
            base = pbase + ci * CH
            pltpu.sync_copy(g2_hbm.at[:, pl.ds(base, CH)], g_v)
            for gidx in range(CH // L):
                gx = g_v[0, pl.ds(gidx * L, L)]
                gy = g_v[1, pl.ds(gidx * L, L)]
                ix = (gx + 1.0) * ((W - 1) / 2.0)
                iy = (gy + 1.0) * ((H - 1) / 2.0)
                ix0 = ix.astype(jnp.int32)
                ix0f = ix0.astype(jnp.float32)
                negx = ix0f > ix
                ix0 = jnp.where(negx, ix0 - 1, ix0)
                ix0f = jnp.where(negx, ix0f - 1.0, ix0f)
                iy0 = iy.astype(jnp.int32)
                iy0f = iy0.astype(jnp.float32)
                negy = iy0f > iy
                iy0 = jnp.where(negy, iy0 - 1, iy0)
                iy0f = jnp.where(negy, iy0f - 1.0, iy0f)
                fx = ix - ix0f
                fy = iy - iy0f
                wx0 = 1.0 - fx
                wy0 = 1.0 - fy
                ix1 = ix0 + 1
                iy1 = iy0 + 1
                mx0 = jnp.where(ix0 >= 0, 1.0, 0.0) * jnp.where(ix0 <= W - 1, 1.0, 0.0)
                mx1 = jnp.where(ix1 >= 0, 1.0, 0.0) * jnp.where(ix1 <= W - 1, 1.0, 0.0)
                my0 = jnp.where(iy0 >= 0, 1.0, 0.0) * jnp.where(iy0 <= H - 1, 1.0, 0.0)
                my1 = jnp.where(iy1 >= 0, 1.0, 0.0) * jnp.where(iy1 <= H - 1, 1.0, 0.0)
                cx0 = jnp.minimum(jnp.maximum(ix0, 0), W - 1)
                cx1 = jnp.minimum(jnp.maximum(ix1, 0), W - 1)
                cy0 = jnp.minimum(jnp.maximum(iy0, 0), H - 1)
                cy1 = jnp.minimum(jnp.maximum(iy1, 0), H - 1)
                nb = nimg * HW
                s = pl.ds(gidx * L, L)
                i00[s] = nb + cy0 * W + cx0
                i01[s] = nb + cy0 * W + cx1
                i10[s] = nb + cy1 * W + cx0
                i11[s] = nb + cy1 * W + cx1
                w00[s] = wy0 * wx0 * (my0 * mx0)
                w01[s] = wy0 * fx * (my0 * mx1)
                w10[s] = fy * wx0 * (my1 * mx0)
                w11[s] = fy * fx * (my1 * mx1)
            pltpu.async_copy(xt_hbm.at[i00], r00, sem)
            pltpu.async_copy(xt_hbm.at[i01], r01, sem)
            pltpu.async_copy(xt_hbm.at[i10], r10, sem)
            pltpu.async_copy(xt_hbm.at[i11], r11, sem)

        def drain_gathers(S):
            (_g, i00, i01, i10, i11, _w0, _w1, _w2, _w3,
             r00, r01, r10, r11, _out, sem, _osem) = S
            pltpu.make_async_copy(xt_hbm.at[i00], r00, sem).wait()
            pltpu.make_async_copy(xt_hbm.at[i01], r01, sem).wait()
            pltpu.make_async_copy(xt_hbm.at[i10], r10, sem).wait()
            pltpu.make_async_copy(xt_hbm.at[i11], r11, sem).wait()

        def out_slice(ci):
            return out_hbm.at[pl.ds(pbase + ci * CH, CH), :]

        def compute_and_write(ci, S, first):
            (_g, _i0, _i1, _i2, _i3, w00, w01, w10, w11,
             r00, r01, r10, r11, out_v, _sem, osem) = S
            drain_gathers(S)

            @pl.when(jnp.logical_not(first))
            def _():
                pltpu.make_async_copy(out_v, out_slice(0), osem).wait()

            for pg in range(CH // L):
                sw = pl.ds(pg * L, L)
                wv00 = w00[sw]
                wv01 = w01[sw]
                wv10 = w10[sw]
                wv11 = w11[sw]

                def pix_body(j, carry, wv00=wv00, wv01=wv01, wv10=wv10,
                             wv11=wv11, pg=pg):
                    b00 = _lane_bcast(wv00, j)
                    b01 = _lane_bcast(wv01, j)
                    b10 = _lane_bcast(wv10, j)
                    b11 = _lane_bcast(wv11, j)
                    i = pg * L + j
                    for cc in range(C // (2 * L)):
                        s = pl.ds(cc * 2 * L, 2 * L)
                        a00, c00 = plsc.unpack(r00[i, s], format=plsc.PackFormat.INTERLEAVED)
                        a01, c01 = plsc.unpack(r01[i, s], format=plsc.PackFormat.INTERLEAVED)
                        a10, c10 = plsc.unpack(r10[i, s], format=plsc.PackFormat.INTERLEAVED)
                        a11, c11 = plsc.unpack(r11[i, s], format=plsc.PackFormat.INTERLEAVED)
                        acc_a = (a00 * b00 + a01 * b01 + a10 * b10 + a11 * b11)
                        acc_c = (c00 * b00 + c01 * b01 + c10 * b10 + c11 * b11)
                        out_v[i, s] = plsc.pack(
                            acc_a, acc_c, format=plsc.PackFormat.INTERLEAVED)
                    return carry

                lax.fori_loop(0, L, pix_body, 0)
            pltpu.async_copy(out_v, out_slice(ci), osem)

        stage_and_fire(0, A)

        def pair_body(cj, carry):
            c0 = 2 * cj
            stage_and_fire(c0 + 1, B)
            compute_and_write(c0, A, cj == 0)
            c2 = jnp.where(c0 + 2 >= NCHUNK, 0, c0 + 2)
            stage_and_fire(c2, A)
            compute_and_write(c0 + 1, B, cj == 0)
            return carry

        lax.fori_loop(0, NPAIR, pair_body, 0)
        drain_gathers(A)
        pltpu.make_async_copy(A[13], out_slice(0), A[15]).wait()
        pltpu.make_async_copy(B[13], out_slice(0), B[15]).wait()

    return grid_sample_sc


_grid_sample_sc = _make_sc_kernel()


def kernel(x, g):
    xt = _transpose_to_rows(x.reshape(N, C, HW)).reshape(P, C)
    gf = g.reshape(P, 2)
    g2 = jnp.stack((gf[:, 0], gf[:, 1]))
    out_t = _grid_sample_sc(xt, g2)
    out = _transpose_to_planes(out_t.reshape(N, HW, C))
    return out.reshape(N, C, HO, WO)


# speedup vs baseline: 1.0482x; 1.0482x over previous
import functools

import jax
import jax.numpy as jnp
from jax import lax
from jax.experimental import pallas as pl
from jax.experimental.pallas import tpu as pltpu
from jax.experimental.pallas import tpu_sc as plsc

N, C, H, W = 4, 192, 224, 224
HO, WO = 224, 224
P = N * HO * WO
HW = H * W
L = 16
NC, NS = 2, 16
NW = NC * NS
PPT = P // NW
TPB = (HO * WO) // PPT
CH = 64
NCHUNK = PPT // CH
NPAIR = NCHUNK // 2
CCH = C // L
TBLK = 3584


def _lane_bcast(v, j):
    idx = jnp.full((L,), j, jnp.int32)
    return lax.gather(
        v, idx[:, None],
        dimension_numbers=lax.GatherDimensionNumbers(
            offset_dims=(), collapsed_slice_dims=(0,), start_index_map=(0,)),
        slice_sizes=(1,), mode=lax.GatherScatterMode.PROMISE_IN_BOUNDS)


RB = TBLK // W


def _eye_bf16():
    ii = lax.broadcasted_iota(jnp.int32, (C, C), 0)
    jj = lax.broadcasted_iota(jnp.int32, (C, C), 1)
    return (ii == jj).astype(jnp.bfloat16)


def _transpose_to_rows(x4):
    def body(x_ref, o_ref):
        eye = _eye_bf16()
        xb = x_ref[0].astype(jnp.bfloat16)
        for r in range(RB):
            y = lax.dot_general(xb[:, r, :], eye, (((0,), (0,)), ((), ())),
                                preferred_element_type=jnp.float32)
            o_ref[pl.ds(r * W, W), :] = y.astype(jnp.bfloat16)
    return pl.pallas_call(
        body,
        grid=(N, HW // TBLK),
        in_specs=[pl.BlockSpec((1, C, RB, W), lambda n, t: (n, 0, t, 0))],
        out_specs=pl.BlockSpec((TBLK, C),
                               lambda n, t: (n * (HW // TBLK) + t, 0)),
        out_shape=jax.ShapeDtypeStruct((P, C), jnp.bfloat16),
    )(x4)


def _transpose_to_planes(y2):
    def body(y_ref, o_ref):
        eye = _eye_bf16()
        for r in range(RB):
            yb = y_ref[pl.ds(r * W, W), :]
            y = lax.dot_general(eye, yb, (((1,), (1,)), ((), ())),
                                preferred_element_type=jnp.float32)
            o_ref[0, :, r, :] = y
    return pl.pallas_call(
        body,
        grid=(N, HW // TBLK),
        in_specs=[pl.BlockSpec((TBLK, C),
                               lambda n, t: (n * (HW // TBLK) + t, 0))],
        out_specs=pl.BlockSpec((1, C, RB, W), lambda n, t: (n, 0, t, 0)),
        out_shape=jax.ShapeDtypeStruct((N, C, H, W), jnp.float32),
    )(y2)


def _make_sc_kernel():
    mesh = plsc.VectorSubcoreMesh(core_axis_name="c", subcore_axis_name="s")

    buf = lambda shape, dt: pltpu.VMEM(shape, dt)
    bufset = lambda: [
        buf((CH, 2), jnp.float32),
        buf((CH,), jnp.int32), buf((CH,), jnp.int32),
        buf((CH,), jnp.int32), buf((CH,), jnp.int32),
        buf((CH,), jnp.float32), buf((CH,), jnp.float32),
        buf((CH,), jnp.float32), buf((CH,), jnp.float32),
        buf((CH, C), jnp.bfloat16), buf((CH, C), jnp.bfloat16),
        buf((CH, C), jnp.bfloat16), buf((CH, C), jnp.bfloat16),
        buf((CH, C), jnp.bfloat16),
        pltpu.SemaphoreType.DMA,
        pltpu.SemaphoreType.DMA,
    ]

    @functools.partial(
        pl.kernel,
        mesh=mesh,
        compiler_params=pltpu.CompilerParams(
            use_tc_tiling_on_sc=False, needs_layout_passes=False),
        out_type=jax.ShapeDtypeStruct((P, C), jnp.bfloat16),
        scratch_types=bufset() + bufset(),
    )
    def grid_sample_sc(xt_hbm, gf_hbm, out_hbm, *scr):
        A, B = scr[:16], scr[16:]
        wid = lax.axis_index("s") * NC + lax.axis_index("c")
        nimg = wid // TPB
        pbase = wid * PPT

        def stage_and_fire(ci, S):
            (g_v, i00, i01, i10, i11, w00, w01, w10, w11,
             r00, r01, r10, r11, _out, sem, _osem) = S
            base = pbase + ci * CH
            pltpu.sync_copy(gf_hbm.at[pl.ds(base, CH), :], g_v)
            lane0 = lax.iota(jnp.int32, L)
            col0 = jnp.zeros((L,), jnp.int32)
            col1 = col0 + 1
            for gidx in range(CH // L):
                row = gidx * L + lane0
                gx = plsc.load_gather(g_v, [row, col0])
                gy = plsc.load_gather(g_v, [row, col1])
                ix = (gx + 1.0) * ((W - 1) / 2.0)
                iy = (gy + 1.0) * ((H - 1) / 2.0)
                ix0 = ix.astype(jnp.int32)
                ix0f = ix0.astype(jnp.float32)
                negx = ix0f > ix
                ix0 = jnp.where(negx, ix0 - 1, ix0)
                ix0f = jnp.where(negx, ix0f - 1.0, ix0f)
                iy0 = iy.astype(jnp.int32)
                iy0f = iy0.astype(jnp.float32)
                negy = iy0f > iy
                iy0 = jnp.where(negy, iy0 - 1, iy0)
                iy0f = jnp.where(negy, iy0f - 1.0, iy0f)
                fx = ix - ix0f
                fy = iy - iy0f
                wx0 = 1.0 - fx
                wy0 = 1.0 - fy
                ix1 = ix0 + 1
                iy1 = iy0 + 1
                mx0 = jnp.where(ix0 >= 0, 1.0, 0.0) * jnp.where(ix0 <= W - 1, 1.0, 0.0)
                mx1 = jnp.where(ix1 >= 0, 1.0, 0.0) * jnp.where(ix1 <= W - 1, 1.0, 0.0)
                my0 = jnp.where(iy0 >= 0, 1.0, 0.0) * jnp.where(iy0 <= H - 1, 1.0, 0.0)
                my1 = jnp.where(iy1 >= 0, 1.0, 0.0) * jnp.where(iy1 <= H - 1, 1.0, 0.0)
                cx0 = jnp.minimum(jnp.maximum(ix0, 0), W - 1)
                cx1 = jnp.minimum(jnp.maximum(ix1, 0), W - 1)
                cy0 = jnp.minimum(jnp.maximum(iy0, 0), H - 1)
                cy1 = jnp.minimum(jnp.maximum(iy1, 0), H - 1)
                nb = nimg * HW
                s = pl.ds(gidx * L, L)
                i00[s] = nb + cy0 * W + cx0
                i01[s] = nb + cy0 * W + cx1
                i10[s] = nb + cy1 * W + cx0
                i11[s] = nb + cy1 * W + cx1
                w00[s] = wy0 * wx0 * (my0 * mx0)
                w01[s] = wy0 * fx * (my0 * mx1)
                w10[s] = fy * wx0 * (my1 * mx0)
                w11[s] = fy * fx * (my1 * mx1)
            pltpu.async_copy(xt_hbm.at[i00], r00, sem)
            pltpu.async_copy(xt_hbm.at[i01], r01, sem)
            pltpu.async_copy(xt_hbm.at[i10], r10, sem)
            pltpu.async_copy(xt_hbm.at[i11], r11, sem)

        def drain_gathers(S):
            (_g, i00, i01, i10, i11, _w0, _w1, _w2, _w3,
             r00, r01, r10, r11, _out, sem, _osem) = S
            pltpu.make_async_copy(xt_hbm.at[i00], r00, sem).wait()
            pltpu.make_async_copy(xt_hbm.at[i01], r01, sem).wait()
            pltpu.make_async_copy(xt_hbm.at[i10], r10, sem).wait()
            pltpu.make_async_copy(xt_hbm.at[i11], r11, sem).wait()

        def out_slice(ci):
            return out_hbm.at[pl.ds(pbase + ci * CH, CH), :]

        def compute_and_write(ci, S, first):
            (_g, _i0, _i1, _i2, _i3, w00, w01, w10, w11,
             r00, r01, r10, r11, out_v, _sem, osem) = S
            drain_gathers(S)

            @pl.when(jnp.logical_not(first))
            def _():
                pltpu.make_async_copy(out_v, out_slice(0), osem).wait()

            for pg in range(CH // L):
                sw = pl.ds(pg * L, L)
                wv00 = w00[sw]
                wv01 = w01[sw]
                wv10 = w10[sw]
                wv11 = w11[sw]

                def pix_body(j, carry, wv00=wv00, wv01=wv01, wv10=wv10,
                             wv11=wv11, pg=pg):
                    b00 = _lane_bcast(wv00, j)
                    b01 = _lane_bcast(wv01, j)
                    b10 = _lane_bcast(wv10, j)
                    b11 = _lane_bcast(wv11, j)
                    i = pg * L + j
                    for cc in range(C // (2 * L)):
                        s = pl.ds(cc * 2 * L, 2 * L)
                        a00, c00 = plsc.unpack(r00[i, s], format=plsc.PackFormat.INTERLEAVED)
                        a01, c01 = plsc.unpack(r01[i, s], format=plsc.PackFormat.INTERLEAVED)
                        a10, c10 = plsc.unpack(r10[i, s], format=plsc.PackFormat.INTERLEAVED)
                        a11, c11 = plsc.unpack(r11[i, s], format=plsc.PackFormat.INTERLEAVED)
                        acc_a = (a00 * b00 + a01 * b01 + a10 * b10 + a11 * b11)
                        acc_c = (c00 * b00 + c01 * b01 + c10 * b10 + c11 * b11)
                        out_v[i, s] = plsc.pack(
                            acc_a, acc_c, format=plsc.PackFormat.INTERLEAVED)
                    return carry

                lax.fori_loop(0, L, pix_body, 0)
            pltpu.async_copy(out_v, out_slice(ci), osem)

        stage_and_fire(0, A)

        def pair_body(cj, carry):
            c0 = 2 * cj
            stage_and_fire(c0 + 1, B)
            compute_and_write(c0, A, cj == 0)
            c2 = jnp.where(c0 + 2 >= NCHUNK, 0, c0 + 2)
            stage_and_fire(c2, A)
            compute_and_write(c0 + 1, B, cj == 0)
            return carry

        lax.fori_loop(0, NPAIR, pair_body, 0)
        drain_gathers(A)
        pltpu.make_async_copy(A[13], out_slice(0), A[15]).wait()
        pltpu.make_async_copy(B[13], out_slice(0), B[15]).wait()

    return grid_sample_sc


_grid_sample_sc = _make_sc_kernel()


def kernel(x, g):
    xt = _transpose_to_rows(x)
    out_t = _grid_sample_sc(xt, g.reshape(P, 2))
    return _transpose_to_planes(out_t)


# speedup vs baseline: 1.0964x; 1.0460x over previous
import functools

import jax
import jax.numpy as jnp
from jax import lax
from jax.experimental import pallas as pl
from jax.experimental.pallas import tpu as pltpu
from jax.experimental.pallas import tpu_sc as plsc

N, C, H, W = 4, 192, 224, 224
HO, WO = 224, 224
P = N * HO * WO
HW = H * W
L = 16
NC, NS = 2, 16
NW = NC * NS
PPT = P // NW
TPB = (HO * WO) // PPT
CH = 64
NCHUNK = PPT // CH
NPAIR = NCHUNK // 2
CCH = C // L
TBLK = 3584


def _lane_bcast(v, j):
    idx = jnp.full((L,), j, jnp.int32)
    return lax.gather(
        v, idx[:, None],
        dimension_numbers=lax.GatherDimensionNumbers(
            offset_dims=(), collapsed_slice_dims=(0,), start_index_map=(0,)),
        slice_sizes=(1,), mode=lax.GatherScatterMode.PROMISE_IN_BOUNDS)


RB = TBLK // W


def _eye_bf16():
    ii = lax.broadcasted_iota(jnp.int32, (C, C), 0)
    jj = lax.broadcasted_iota(jnp.int32, (C, C), 1)
    return (ii == jj).astype(jnp.bfloat16)


def _transpose_to_rows(x4):
    def body(x_ref, o_ref):
        eye = _eye_bf16()
        xb = x_ref[0].astype(jnp.bfloat16)
        for r in range(RB):
            y = lax.dot_general(xb[:, r, :], eye, (((0,), (0,)), ((), ())),
                                preferred_element_type=jnp.float32)
            o_ref[pl.ds(r * W, W), :] = y.astype(jnp.bfloat16)
    return pl.pallas_call(
        body,
        grid=(N, HW // TBLK),
        in_specs=[pl.BlockSpec((1, C, RB, W), lambda n, t: (n, 0, t, 0))],
        out_specs=pl.BlockSpec((TBLK, C),
                               lambda n, t: (n * (HW // TBLK) + t, 0)),
        out_shape=jax.ShapeDtypeStruct((P, C), jnp.bfloat16),
    )(x4)


def _transpose_to_planes(y2):
    def body(y_ref, o_ref):
        eye = _eye_bf16()
        for r in range(RB):
            yb = y_ref[pl.ds(r * W, W), :]
            y = lax.dot_general(eye, yb, (((1,), (1,)), ((), ())),
                                preferred_element_type=jnp.float32)
            o_ref[0, :, r, :] = y
    return pl.pallas_call(
        body,
        grid=(N, HW // TBLK),
        in_specs=[pl.BlockSpec((TBLK, C),
                               lambda n, t: (n * (HW // TBLK) + t, 0))],
        out_specs=pl.BlockSpec((1, C, RB, W), lambda n, t: (n, 0, t, 0)),
        out_shape=jax.ShapeDtypeStruct((N, C, H, W), jnp.float32),
    )(y2)


def _make_sc_kernel():
    mesh = plsc.VectorSubcoreMesh(core_axis_name="c", subcore_axis_name="s")

    buf = lambda shape, dt: pltpu.VMEM(shape, dt)
    bufset = lambda: [
        buf((1, 2 * CH), jnp.float32),
        buf((CH,), jnp.int32), buf((CH,), jnp.int32),
        buf((CH,), jnp.int32), buf((CH,), jnp.int32),
        buf((CH,), jnp.float32), buf((CH,), jnp.float32),
        buf((CH,), jnp.float32), buf((CH,), jnp.float32),
        buf((CH, C), jnp.bfloat16), buf((CH, C), jnp.bfloat16),
        buf((CH, C), jnp.bfloat16), buf((CH, C), jnp.bfloat16),
        buf((CH, C), jnp.bfloat16),
        pltpu.SemaphoreType.DMA,
        pltpu.SemaphoreType.DMA,
    ]

    @functools.partial(
        pl.kernel,
        mesh=mesh,
        compiler_params=pltpu.CompilerParams(
            use_tc_tiling_on_sc=False, needs_layout_passes=False),
        out_type=jax.ShapeDtypeStruct((P, C), jnp.bfloat16),
        scratch_types=bufset() + bufset(),
    )
    def grid_sample_sc(xt_hbm, gf_hbm, out_hbm, *scr):
        A, B = scr[:16], scr[16:]
        wid = lax.axis_index("s") * NC + lax.axis_index("c")
        nimg = wid // TPB
        pbase = wid * PPT

        def stage_and_fire(ci, S):
            (g_v, i00, i01, i10, i11, w00, w01, w10, w11,
             r00, r01, r10, r11, _out, sem, _osem) = S
            grow = (pbase * 2) // (2 * CH) + ci
            pltpu.sync_copy(gf_hbm.at[pl.ds(grow, 1), :], g_v)
            lane0 = lax.iota(jnp.int32, L)
            row0 = jnp.zeros((L,), jnp.int32)
            for gidx in range(CH // L):
                colx = (gidx * L + lane0) * 2
                gx = plsc.load_gather(g_v, [row0, colx])
                gy = plsc.load_gather(g_v, [row0, colx + 1])
                ix = (gx + 1.0) * ((W - 1) / 2.0)
                iy = (gy + 1.0) * ((H - 1) / 2.0)
                ix0 = ix.astype(jnp.int32)
                ix0f = ix0.astype(jnp.float32)
                negx = ix0f > ix
                ix0 = jnp.where(negx, ix0 - 1, ix0)
                ix0f = jnp.where(negx, ix0f - 1.0, ix0f)
                iy0 = iy.astype(jnp.int32)
                iy0f = iy0.astype(jnp.float32)
                negy = iy0f > iy
                iy0 = jnp.where(negy, iy0 - 1, iy0)
                iy0f = jnp.where(negy, iy0f - 1.0, iy0f)
                fx = ix - ix0f
                fy = iy - iy0f
                wx0 = 1.0 - fx
                wy0 = 1.0 - fy
                ix1 = ix0 + 1
                iy1 = iy0 + 1
                mx0 = jnp.where(ix0 >= 0, 1.0, 0.0) * jnp.where(ix0 <= W - 1, 1.0, 0.0)
                mx1 = jnp.where(ix1 >= 0, 1.0, 0.0) * jnp.where(ix1 <= W - 1, 1.0, 0.0)
                my0 = jnp.where(iy0 >= 0, 1.0, 0.0) * jnp.where(iy0 <= H - 1, 1.0, 0.0)
                my1 = jnp.where(iy1 >= 0, 1.0, 0.0) * jnp.where(iy1 <= H - 1, 1.0, 0.0)
                cx0 = jnp.minimum(jnp.maximum(ix0, 0), W - 1)
                cx1 = jnp.minimum(jnp.maximum(ix1, 0), W - 1)
                cy0 = jnp.minimum(jnp.maximum(iy0, 0), H - 1)
                cy1 = jnp.minimum(jnp.maximum(iy1, 0), H - 1)
                nb = nimg * HW
                s = pl.ds(gidx * L, L)
                i00[s] = nb + cy0 * W + cx0
                i01[s] = nb + cy0 * W + cx1
                i10[s] = nb + cy1 * W + cx0
                i11[s] = nb + cy1 * W + cx1
                w00[s] = wy0 * wx0 * (my0 * mx0)
                w01[s] = wy0 * fx * (my0 * mx1)
                w10[s] = fy * wx0 * (my1 * mx0)
                w11[s] = fy * fx * (my1 * mx1)
            pltpu.async_copy(xt_hbm.at[i00], r00, sem)
            pltpu.async_copy(xt_hbm.at[i01], r01, sem)
            pltpu.async_copy(xt_hbm.at[i10], r10, sem)
            pltpu.async_copy(xt_hbm.at[i11], r11, sem)

        def drain_gathers(S):
            (_g, i00, i01, i10, i11, _w0, _w1, _w2, _w3,
             r00, r01, r10, r11, _out, sem, _osem) = S
            pltpu.make_async_copy(xt_hbm.at[i00], r00, sem).wait()
            pltpu.make_async_copy(xt_hbm.at[i01], r01, sem).wait()
            pltpu.make_async_copy(xt_hbm.at[i10], r10, sem).wait()
            pltpu.make_async_copy(xt_hbm.at[i11], r11, sem).wait()

        def out_slice(ci):
            return out_hbm.at[pl.ds(pbase + ci * CH, CH), :]

        def compute_and_write(ci, S, first):
            (_g, _i0, _i1, _i2, _i3, w00, w01, w10, w11,
             r00, r01, r10, r11, out_v, _sem, osem) = S
            drain_gathers(S)

            @pl.when(jnp.logical_not(first))
            def _():
                pltpu.make_async_copy(out_v, out_slice(0), osem).wait()

            for pg in range(CH // L):
                sw = pl.ds(pg * L, L)
                wv00 = w00[sw]
                wv01 = w01[sw]
                wv10 = w10[sw]
                wv11 = w11[sw]

                def pix_body(j, carry, wv00=wv00, wv01=wv01, wv10=wv10,
                             wv11=wv11, pg=pg):
                    b00 = _lane_bcast(wv00, j)
                    b01 = _lane_bcast(wv01, j)
                    b10 = _lane_bcast(wv10, j)
                    b11 = _lane_bcast(wv11, j)
                    i = pg * L + j
                    for cc in range(C // (2 * L)):
                        s = pl.ds(cc * 2 * L, 2 * L)
                        a00, c00 = plsc.unpack(r00[i, s], format=plsc.PackFormat.INTERLEAVED)
                        a01, c01 = plsc.unpack(r01[i, s], format=plsc.PackFormat.INTERLEAVED)
                        a10, c10 = plsc.unpack(r10[i, s], format=plsc.PackFormat.INTERLEAVED)
                        a11, c11 = plsc.unpack(r11[i, s], format=plsc.PackFormat.INTERLEAVED)
                        acc_a = (a00 * b00 + a01 * b01 + a10 * b10 + a11 * b11)
                        acc_c = (c00 * b00 + c01 * b01 + c10 * b10 + c11 * b11)
                        out_v[i, s] = plsc.pack(
                            acc_a, acc_c, format=plsc.PackFormat.INTERLEAVED)
                    return carry

                lax.fori_loop(0, L, pix_body, 0)
            pltpu.async_copy(out_v, out_slice(ci), osem)

        stage_and_fire(0, A)

        def pair_body(cj, carry):
            c0 = 2 * cj
            stage_and_fire(c0 + 1, B)
            compute_and_write(c0, A, cj == 0)
            c2 = jnp.where(c0 + 2 >= NCHUNK, 0, c0 + 2)
            stage_and_fire(c2, A)
            compute_and_write(c0 + 1, B, cj == 0)
            return carry

        lax.fori_loop(0, NPAIR, pair_body, 0)
        drain_gathers(A)
        pltpu.make_async_copy(A[13], out_slice(0), A[15]).wait()
        pltpu.make_async_copy(B[13], out_slice(0), B[15]).wait()

    return grid_sample_sc


_grid_sample_sc = _make_sc_kernel()


def kernel(x, g):
    xt = _transpose_to_rows(x)
    gf = g.reshape(2 * P).reshape((2 * P) // (2 * CH), 2 * CH)
    out_t = _grid_sample_sc(xt, gf)
    return _transpose_to_planes(out_t)
